# baseline (speedup 1.0000x reference)
import functools

import jax
import jax.numpy as jnp
from jax import lax
from jax.experimental import pallas as pl
from jax.experimental.pallas import tpu as pltpu
from jax.experimental.pallas import tpu_sc as plsc

_NC = 2
_NS = 16
_NW = _NC * _NS
_L = 16

_SC_PARAMS = pltpu.CompilerParams(needs_layout_passes=False)


def _sc_scatter_body(ei_hbm, et_hbm, out_hbm, src_v, dst_v, typ_v, tbl_v,
                     *, epw, slots, S):
    wid = lax.axis_index("s") * _NC + lax.axis_index("c")
    base = wid * epw
    pltpu.sync_copy(ei_hbm.at[0, pl.ds(base, epw)], src_v)
    pltpu.sync_copy(ei_hbm.at[1, pl.ds(base, epw)], dst_v)
    pltpu.sync_copy(et_hbm.at[pl.ds(base, epw)], typ_v)

    neg1 = jnp.full((_L,), -1, jnp.int32)

    def init_body(i):
        tbl_v[pl.ds(i * _L, _L)] = neg1

    plsc.parallel_loop(0, slots // _L, 1, unroll=8)(init_body)

    iot = lax.iota(jnp.int32, _L)

    def edge_body(i, _):
        for u in range(4):
            off = (i * 4 + u) * _L
            sl = pl.ds(off, _L)
            fl = src_v[sl] * S + dst_v[sl]
            pk = jnp.bitwise_or((base + off + iot) << 4, typ_v[sl])
            _, keep = plsc.scan_count(fl)
            plsc.store_scatter(tbl_v, [fl], pk, mask=keep)
        return 0

    lax.fori_loop(0, epw // (_L * 4), edge_body, 0)

    pltpu.sync_copy(tbl_v, out_hbm.at[wid])


def _sc_scatter(edge_index, edge_type, S):
    E = edge_type.shape[0]
    epw = E // _NW
    slots = S * S
    mesh = plsc.VectorSubcoreMesh(core_axis_name="c", subcore_axis_name="s")
    body = functools.partial(_sc_scatter_body, epw=epw, slots=slots, S=S)
    return pl.kernel(
        body,
        out_type=jax.ShapeDtypeStruct((_NW, slots), jnp.int32),
        mesh=mesh,
        scratch_types=[
            pltpu.VMEM((epw,), jnp.int32),
            pltpu.VMEM((epw,), jnp.int32),
            pltpu.VMEM((epw,), jnp.int32),
            pltpu.VMEM((slots,), jnp.int32),
        ],
        compiler_params=_SC_PARAMS,
    )(edge_index, edge_type)


def _zeros_kernel(out_ref):
    out_ref[...] = jnp.zeros_like(out_ref)


def _zeros(B, H, S):
    rows = B * H * S
    return pl.pallas_call(
        _zeros_kernel,
        grid=(B,),
        out_specs=pl.BlockSpec((rows // B, S), lambda b: (b, 0)),
        out_shape=jax.ShapeDtypeStruct((rows, S), jnp.float32),
    )()


def _sc_vrows_body(tbl_ref, emb_ref, v_hbm, tseg_v, emb_v, mseg_v,
                   vrows, sem, *, S, H, spw):
    wid = lax.axis_index("s") * _NC + lax.axis_index("c")
    seg = spw * S

    pltpu.sync_copy(emb_ref, emb_v)

    copies = []
    for k in range(_NW):
        copies.append(pltpu.async_copy(
            tbl_ref.at[k, pl.ds(wid * seg, seg)], tseg_v.at[k], sem))
    for c in copies:
        c.wait()

    def m_body(dv):
        base = dv * _L
        p = tseg_v[0, pl.ds(base, _L)]
        for k in range(1, _NW):
            p = jnp.maximum(p, tseg_v[k, pl.ds(base, _L)])
        mseg_v[pl.ds(base, _L)] = jnp.where(
            p >= 0, jnp.bitwise_and(p, 15) << 4, H * H + _L)

    plsc.parallel_loop(0, seg // _L, 1)(m_body)

    pending = []
    for i in range(spw):
        s = wid * spw + i
        vrow_v = vrows[i]

        def g_body(dv, i=i, vrow_v=vrow_v):
            tb = mseg_v[pl.ds(i * S + dv * _L, _L)]
            for h in range(H):
                g = plsc.load_gather(emb_v, [tb + h])
                vrow_v[h, pl.ds(dv * _L, _L)] = g

        plsc.parallel_loop(0, S // _L, 1)(g_body)
        pending.append(pltpu.async_copy(vrow_v, v_hbm.at[s], sem))
    for p in pending:
        p.wait()


def _sc_vrows(tables, embf, S, H):
    spw = S // _NW
    mesh = plsc.VectorSubcoreMesh(core_axis_name="c", subcore_axis_name="s")
    body = functools.partial(_sc_vrows_body, S=S, H=H, spw=spw)
    return pl.kernel(
        body,
        out_type=jax.ShapeDtypeStruct((S, H, S), jnp.float32),
        mesh=mesh,
        scratch_types=[
            pltpu.VMEM((_NW, spw * S), jnp.int32),
            pltpu.VMEM((H * H + 2 * _L,), jnp.float32),
            pltpu.VMEM((spw * S,), jnp.int32),
            [pltpu.VMEM((H, S), jnp.float32) for _ in range(spw)],
            pltpu.SemaphoreType.DMA,
        ],
        compiler_params=_SC_PARAMS,
    )(tables, embf)


def _sc_scatter_rows_body(v_ref, bv_ref, out_ref, vblk_v, bv_v, sem,
                          *, S, H, spw):
    wid = lax.axis_index("s") * _NC + lax.axis_index("c")

    pltpu.sync_copy(bv_ref.at[pl.ds(wid * spw, _L)], bv_v)
    pltpu.sync_copy(v_ref.at[pl.ds(wid * spw, spw)], vblk_v)

    bvv = bv_v[pl.ds(0, _L)]
    iot = lax.iota(jnp.int32, _L)

    pending = []
    for i in range(spw):
        s = wid * spw + i
        b = bvv[i]
        idx = b * (H * S) + iot * S + s
        pending.append(pltpu.async_copy(vblk_v.at[i], out_ref.at[idx], sem))
    for p in pending:
        p.wait()


def _sc_rows(vrows, bv_pad, buf, S, H):
    spw = S // _NW
    mesh = plsc.VectorSubcoreMesh(core_axis_name="c", subcore_axis_name="s")
    body = functools.partial(_sc_scatter_rows_body, S=S, H=H, spw=spw)

    def inner(refs):
        v_ref, bv_ref, out_ref = refs

        @pl.core_map(
            mesh, compiler_params=_SC_PARAMS,
            scratch_shapes=[
                pltpu.VMEM((spw, H, S), jnp.float32),
                pltpu.VMEM((_L,), jnp.int32),
                pltpu.SemaphoreType.DMA,
            ],
        )
        def _(vblk_v, bv_v, sem):
            body(v_ref, bv_ref, out_ref, vblk_v, bv_v, sem)

    _, _, out = pl.run_state(inner)((vrows, bv_pad, buf))
    return out


def kernel(edge_index, edge_type, batch_vec, batch_size, max_seq_len,
           graph_node_offsets, edge_embedding):
    E = edge_type.shape[0]
    S = batch_vec.shape[0]
    B = graph_node_offsets.shape[0]
    H = edge_embedding.shape[1]

    tables = _sc_scatter(edge_index, edge_type, S)
    buf = _zeros(B, H, S)

    bv_pad = jnp.pad(batch_vec, (0, _L))
    embf = jnp.pad(edge_embedding[:H].reshape(-1), (0, 2 * _L))

    vrows = _sc_vrows(tables, embf, S, H)
    out = _sc_rows(vrows, bv_pad, buf, S, H)
    return out.reshape(B, H, S, S)


# speedup vs baseline: 1.0855x; 1.0855x over previous
import functools

import jax
import jax.numpy as jnp
from jax import lax
from jax.experimental import pallas as pl
from jax.experimental.pallas import tpu as pltpu
from jax.experimental.pallas import tpu_sc as plsc

_NC = 2
_NS = 16
_NW = _NC * _NS
_L = 16

_SC_PARAMS = pltpu.CompilerParams(needs_layout_passes=False)


def _sc_scatter_body(ei_hbm, et_hbm, out_hbm, src_v, dst_v, typ_v, tbl_v,
                     *, epw, slots, S):
    wid = lax.axis_index("s") * _NC + lax.axis_index("c")
    base = wid * epw
    pltpu.sync_copy(ei_hbm.at[0, pl.ds(base, epw)], src_v)
    pltpu.sync_copy(ei_hbm.at[1, pl.ds(base, epw)], dst_v)
    pltpu.sync_copy(et_hbm.at[pl.ds(base, epw)], typ_v)

    neg1 = jnp.full((_L,), -1, jnp.int32)

    def init_body(i):
        tbl_v[pl.ds(i * _L, _L)] = neg1

    plsc.parallel_loop(0, slots // _L, 1, unroll=8)(init_body)

    iot = lax.iota(jnp.int32, _L)

    def edge_body(i, _):
        for u in range(4):
            off = (i * 4 + u) * _L
            sl = pl.ds(off, _L)
            fl = src_v[sl] * S + dst_v[sl]
            pk = jnp.bitwise_or((base + off + iot) << 4, typ_v[sl])
            _, keep = plsc.scan_count(fl)
            plsc.store_scatter(tbl_v, [fl], pk, mask=keep)
        return 0

    lax.fori_loop(0, epw // (_L * 4), edge_body, 0)

    pltpu.sync_copy(tbl_v, out_hbm.at[wid])


def _sc_scatter(edge_index, edge_type, S):
    E = edge_type.shape[0]
    epw = E // _NW
    slots = S * S
    mesh = plsc.VectorSubcoreMesh(core_axis_name="c", subcore_axis_name="s")
    body = functools.partial(_sc_scatter_body, epw=epw, slots=slots, S=S)
    return pl.kernel(
        body,
        out_type=jax.ShapeDtypeStruct((_NW, slots), jnp.int32),
        mesh=mesh,
        scratch_types=[
            pltpu.VMEM((epw,), jnp.int32),
            pltpu.VMEM((epw,), jnp.int32),
            pltpu.VMEM((epw,), jnp.int32),
            pltpu.VMEM((slots,), jnp.int32),
        ],
        compiler_params=_SC_PARAMS,
    )(edge_index, edge_type)


def _zeros_kernel(out_ref):
    out_ref[...] = jnp.zeros_like(out_ref)


def _zeros(B, H, S):
    rows = B * H * S
    return pl.pallas_call(
        _zeros_kernel,
        grid=(B,),
        out_specs=pl.BlockSpec((rows // B, S), lambda b: (b, 0)),
        out_shape=jax.ShapeDtypeStruct((rows, S), jnp.float32),
    )()


def _sc_rows_body(tbl_ref, bv_ref, emb_ref, out_ref, tseg_v, emb_v, bv_v,
                  mseg_v, vrows, sem, *, S, H, spw):
    wid = lax.axis_index("s") * _NC + lax.axis_index("c")
    seg = spw * S

    pltpu.sync_copy(emb_ref, emb_v)
    pltpu.sync_copy(bv_ref.at[pl.ds(wid * spw, _L)], bv_v)

    copies = []
    for k in range(_NW):
        copies.append(pltpu.async_copy(
            tbl_ref.at[k, pl.ds(wid * seg, seg)], tseg_v.at[k], sem))
    for c in copies:
        c.wait()

    def m_body(dv):
        base = dv * _L
        p = tseg_v[0, pl.ds(base, _L)]
        for k in range(1, _NW):
            p = jnp.maximum(p, tseg_v[k, pl.ds(base, _L)])
        mseg_v[pl.ds(base, _L)] = jnp.where(
            p >= 0, jnp.bitwise_and(p, 15) << 4, H * H + _L)

    plsc.parallel_loop(0, seg // _L, 1)(m_body)

    bvv = bv_v[pl.ds(0, _L)]
    iot = lax.iota(jnp.int32, _L)

    pending = []
    for i in range(spw):
        s = wid * spw + i
        b = bvv[i]
        vrow_v = vrows[i]

        def g_body(dv, i=i, vrow_v=vrow_v):
            tb = mseg_v[pl.ds(i * S + dv * _L, _L)]
            for h in range(H):
                g = plsc.load_gather(emb_v, [tb + h])
                vrow_v[h, pl.ds(dv * _L, _L)] = g

        plsc.parallel_loop(0, S // _L, 1)(g_body)

        idx = b * (H * S) + iot * S + s
        pending.append(pltpu.async_copy(vrow_v, out_ref.at[idx], sem))
    for p in pending:
        p.wait()


def _sc_rows(tables, bv_pad, embf, buf, S, H):
    spw = S // _NW
    mesh = plsc.VectorSubcoreMesh(core_axis_name="c", subcore_axis_name="s")
    body = functools.partial(_sc_rows_body, S=S, H=H, spw=spw)

    def inner(refs):
        tbl_ref, bv_ref, emb_ref, out_ref = refs

        @pl.core_map(
            mesh, compiler_params=_SC_PARAMS,
            scratch_shapes=[
                pltpu.VMEM((_NW, spw * S), jnp.int32),
                pltpu.VMEM((H * H + 2 * _L,), jnp.float32),
                pltpu.VMEM((_L,), jnp.int32),
                pltpu.VMEM((spw * S,), jnp.int32),
                [pltpu.VMEM((H, S), jnp.float32) for _ in range(spw)],
                pltpu.SemaphoreType.DMA,
            ],
        )
        def _(tseg_v, emb_v, bv_v, mseg_v, vrows, sem):
            body(tbl_ref, bv_ref, emb_ref, out_ref, tseg_v, emb_v, bv_v,
                 mseg_v, vrows, sem)

    _, _, _, out = pl.run_state(inner)((tables, bv_pad, embf, buf))
    return out


def kernel(edge_index, edge_type, batch_vec, batch_size, max_seq_len,
           graph_node_offsets, edge_embedding):
    E = edge_type.shape[0]
    S = batch_vec.shape[0]
    B = graph_node_offsets.shape[0]
    H = edge_embedding.shape[1]

    tables = _sc_scatter(edge_index, edge_type, S)
    buf = _zeros(B, H, S)

    bv_pad = jnp.pad(batch_vec, (0, _L))
    embf = jnp.pad(edge_embedding[:H].reshape(-1), (0, 2 * _L))

    out = _sc_rows(tables, bv_pad, embf, buf, S, H)
    return out.reshape(B, H, S, S)
